# baseline (speedup 1.0000x reference)
import functools

import jax
import jax.numpy as jnp
from jax import lax
from jax.experimental import pallas as pl
from jax.experimental.pallas import tpu as pltpu
from jax.experimental.pallas import tpu_sc as plsc

_NUM_EMBED = 8192
_EMBED_DIM = 256
_BETA = 0.25
_TOKENS = 16384

_TM = 512
_NW = 32
_TOK_PER_W = _TOKENS // _NW
_CHUNK = 128
_NCHUNK = _TOK_PER_W // _CHUNK


def _normalize_body(w_ref, out_ref, wn_ref):
    w = w_ref[...]
    n = jnp.sqrt(jnp.sum(w * w, axis=1, keepdims=True))
    out_ref[...] = (w / jnp.maximum(n, 1e-12)).astype(jnp.bfloat16)
    wn_ref[...] = n


def _argmax_body(z_ref, nw_ref, idx_ref, sn_ref, a_ref, acc_ref):
    i = pl.program_id(0)
    zc = z_ref[...].reshape(_EMBED_DIM, _TM)
    ssum = jnp.sum(zc * zc, axis=0, keepdims=True)
    n = jnp.sqrt(ssum)
    zn = (zc / jnp.maximum(n, 1e-12)).astype(jnp.bfloat16)
    scores = lax.dot_general(
        zn, nw_ref[...], (((0,), (1,)), ((), ())),
        preferred_element_type=jnp.float32)
    val = scores[:, 0:128]
    tid = jnp.zeros(val.shape, jnp.float32)
    for t in range(1, _NUM_EMBED // 128):
        tile = scores[:, t * 128:(t + 1) * 128]
        cond = tile >= val
        val = jnp.maximum(val, tile)
        tid = jnp.where(cond, jnp.float32(t), tid)
    m = jnp.max(val, axis=1, keepdims=True)
    lane = lax.broadcasted_iota(jnp.int32, val.shape, 1).astype(jnp.float32)
    g = tid * jnp.float32(128.0) + lane
    best = jnp.max(jnp.where(val == m, g, jnp.float32(-1.0)), axis=1)
    idx_ref[0, 0, :] = best.astype(jnp.int32)
    sn_ref[0, 0, :] = jnp.squeeze(m, 1) * jnp.squeeze(n, 0)
    at = jnp.sum(ssum)

    @pl.when(i == 0)
    def _():
        acc_ref[0] = at

    @pl.when(i > 0)
    def _():
        acc_ref[0] = acc_ref[0] + at

    @pl.when(i == pl.num_programs(0) - 1)
    def _():
        a_ref[...] = jnp.reshape(acc_ref[0], (1, 1))


def _sc_body(w_hbm, idx_hbm, sn_hbm,
             zq_hbm, cnt_hbm, wsum_hbm,
             idx_v, sn_v, rows_v, ones_v, zeros_v,
             cnt_sh, wsum_sh, sem, addsem):
    cid = lax.axis_index("c")
    sid = lax.axis_index("s")
    wid = cid * 16 + sid
    base = wid * _TOK_PER_W

    pltpu.sync_copy(idx_hbm.at[pl.ds(wid * _NCHUNK, _NCHUNK)], idx_v)
    pltpu.sync_copy(sn_hbm.at[pl.ds(wid * _NCHUNK, _NCHUNK)], sn_v)

    def _zero(k, c):
        zeros_v[pl.ds(k * 16, 16)] = jnp.zeros((16,), jnp.float32)
        return c
    lax.fori_loop(0, (_NUM_EMBED // 16) // 16, _zero, 0)
    sl = pl.ds(sid * (_NUM_EMBED // 16), _NUM_EMBED // 16)
    pltpu.sync_copy(zeros_v, cnt_sh.at[sl])
    pltpu.sync_copy(zeros_v, wsum_sh.at[sl])

    for k in range(_CHUNK // 16):
        ones_v[pl.ds(k * 16, 16)] = jnp.ones((16,), jnp.float32)

    plsc.subcore_barrier()

    adds = [pltpu.async_copy(ones_v, cnt_sh.at[idx_v.at[j]], addsem,
                             add=True)
            for j in range(_NCHUNK)]
    adds += [pltpu.async_copy(sn_v.at[j], wsum_sh.at[idx_v.at[j]], addsem,
                              add=True)
             for j in range(_NCHUNK)]

    for j in range(_NCHUNK):
        pltpu.async_copy(w_hbm.at[idx_v.at[j]], rows_v, sem).wait()
        pltpu.sync_copy(rows_v, zq_hbm.at[pl.ds(base + j * _CHUNK, _CHUNK)])

    for a in adds:
        a.wait()
    plsc.subcore_barrier()

    @pl.when(sid == 0)
    def _():
        pltpu.sync_copy(cnt_sh, cnt_hbm.at[cid])
        pltpu.sync_copy(wsum_sh, wsum_hbm.at[cid])


def _finalize_body(cnt_ref, wsum_ref, wn_ref, a_ref, loss_ref, ppx_ref):
    a = jnp.sum(a_ref[...])
    wn = wn_ref[...]
    wsum = wsum_ref[0:1, :] + wsum_ref[1:2, :]
    cntf = cnt_ref[0:1, :] + cnt_ref[1:2, :]
    bsum = jnp.sum(wsum * wn)
    csum = jnp.sum(cntf * wn * wn)
    mval = (a - 2.0 * bsum + csum) / jnp.float32(_TOKENS * _EMBED_DIM)
    loss_ref[...] = jnp.reshape(jnp.float32(_BETA) * mval + mval, (1, 1))
    p = cntf * jnp.float32(1.0 / _TOKENS)
    ent = jnp.sum(p * jnp.log(p + 1e-10))
    ppx_ref[...] = jnp.reshape(jnp.exp(-ent), (1, 1))


@functools.lru_cache(maxsize=1)
def _make_sc_gather_hist():
    return pl.kernel(
        _sc_body,
        out_type=(jax.ShapeDtypeStruct((_TOKENS, _EMBED_DIM), jnp.float32),
                  jax.ShapeDtypeStruct((2, _NUM_EMBED), jnp.float32),
                  jax.ShapeDtypeStruct((2, _NUM_EMBED), jnp.float32)),
        mesh=plsc.VectorSubcoreMesh(core_axis_name="c", subcore_axis_name="s",
                                    num_cores=2, num_subcores=16),
        scratch_types=[
            pltpu.VMEM((_NCHUNK, _CHUNK), jnp.int32),
            pltpu.VMEM((_NCHUNK, _CHUNK), jnp.float32),
            pltpu.VMEM((_CHUNK, _EMBED_DIM), jnp.float32),
            pltpu.VMEM((_CHUNK,), jnp.float32),
            pltpu.VMEM((_NUM_EMBED // 16,), jnp.float32),
            pltpu.VMEM_SHARED((_NUM_EMBED,), jnp.float32),
            pltpu.VMEM_SHARED((_NUM_EMBED,), jnp.float32),
            pltpu.SemaphoreType.DMA,
            pltpu.SemaphoreType.DMA,
        ],
    )


def kernel(z, weight):
    b, c, h, w = z.shape
    z3 = z.reshape(b, c, h * w)

    normed, wncol = pl.pallas_call(
        _normalize_body,
        out_shape=[jax.ShapeDtypeStruct((_NUM_EMBED, _EMBED_DIM),
                                        jnp.bfloat16),
                   jax.ShapeDtypeStruct((_NUM_EMBED, 1), jnp.float32)],
    )(weight)

    n_tiles = _TOKENS // _TM
    idx3, sn3, asum = pl.pallas_call(
        _argmax_body,
        grid=(n_tiles,),
        in_specs=[
            pl.BlockSpec((1, _EMBED_DIM, _TM), lambda i: (i // 2, 0, i % 2)),
            pl.BlockSpec((_NUM_EMBED, _EMBED_DIM), lambda i: (0, 0)),
        ],
        out_specs=[
            pl.BlockSpec((1, 1, _TM), lambda i: (i, 0, 0)),
            pl.BlockSpec((1, 1, _TM), lambda i: (i, 0, 0)),
            pl.BlockSpec((1, 1), lambda i: (0, 0)),
        ],
        out_shape=[
            jax.ShapeDtypeStruct((n_tiles, 1, _TM), jnp.int32),
            jax.ShapeDtypeStruct((n_tiles, 1, _TM), jnp.float32),
            jax.ShapeDtypeStruct((1, 1), jnp.float32),
        ],
        scratch_shapes=[pltpu.SMEM((1,), jnp.float32)],
    )(z3, normed)
    idx = idx3.reshape(_TOKENS)

    zq, cnt, wsum = _make_sc_gather_hist()(
        weight, idx.reshape(128, 128), sn3.reshape(128, 128))

    loss, ppx = pl.pallas_call(
        _finalize_body,
        out_shape=[jax.ShapeDtypeStruct((1, 1), jnp.float32),
                   jax.ShapeDtypeStruct((1, 1), jnp.float32)],
    )(cnt, wsum, wncol.reshape(1, _NUM_EMBED), asum)

    z_q_out = zq.reshape(b, h, w, c).transpose(0, 3, 1, 2)
    return z_q_out, loss[0, 0], ppx[0, 0], idx


# speedup vs baseline: 1.1086x; 1.1086x over previous
import functools

import jax
import jax.numpy as jnp
from jax import lax
from jax.experimental import pallas as pl
from jax.experimental.pallas import tpu as pltpu
from jax.experimental.pallas import tpu_sc as plsc

_NUM_EMBED = 8192
_EMBED_DIM = 256
_BETA = 0.25
_TOKENS = 16384

_TM = 512
_NW = 32
_TOK_PER_W = _TOKENS // _NW
_CHUNK = 128
_NCHUNK = _TOK_PER_W // _CHUNK


def _normalize_body(w_ref, out_ref, wn_ref):
    w = w_ref[...]
    n = jnp.sqrt(jnp.sum(w * w, axis=1, keepdims=True))
    out_ref[...] = (w / jnp.maximum(n, 1e-12)).astype(jnp.bfloat16)
    wn_ref[...] = n


def _argmax_body(z_ref, nw_ref, idx_ref, sn_ref, a_ref, acc_ref):
    i = pl.program_id(0)
    zt = z_ref[...]
    ssum = jnp.sum(zt * zt, axis=1, keepdims=True)
    n = jnp.sqrt(ssum)
    zn = (zt / jnp.maximum(n, 1e-12)).astype(jnp.bfloat16)
    scores = lax.dot_general(
        zn, nw_ref[...], (((1,), (1,)), ((), ())),
        preferred_element_type=jnp.float32)
    val = scores[:, 0:128]
    tid = jnp.zeros(val.shape, jnp.float32)
    for t in range(1, _NUM_EMBED // 128):
        tile = scores[:, t * 128:(t + 1) * 128]
        cond = tile >= val
        val = jnp.maximum(val, tile)
        tid = jnp.where(cond, jnp.float32(t), tid)
    m = jnp.max(val, axis=1, keepdims=True)
    lane = lax.broadcasted_iota(jnp.int32, val.shape, 1).astype(jnp.float32)
    g = tid * jnp.float32(128.0) + lane
    best = jnp.max(jnp.where(val == m, g, jnp.float32(-1.0)), axis=1)
    idx_ref[0, 0, :] = best.astype(jnp.int32)
    sn_ref[0, 0, :] = jnp.squeeze(m * n, 1)
    at = jnp.sum(ssum)

    @pl.when(i == 0)
    def _():
        acc_ref[0] = at

    @pl.when(i > 0)
    def _():
        acc_ref[0] = acc_ref[0] + at

    @pl.when(i == pl.num_programs(0) - 1)
    def _():
        a_ref[...] = jnp.reshape(acc_ref[0], (1, 1))


def _sc_body(w_hbm, idx_hbm, sn_hbm,
             zq_hbm, cnt_hbm, wsum_hbm,
             idx_v, sn_v, rows_v, ones_v, zeros_v,
             cnt_sh, wsum_sh, sem, addsem):
    cid = lax.axis_index("c")
    sid = lax.axis_index("s")
    wid = cid * 16 + sid
    base = wid * _TOK_PER_W

    pltpu.sync_copy(idx_hbm.at[pl.ds(wid * _NCHUNK, _NCHUNK)], idx_v)
    pltpu.sync_copy(sn_hbm.at[pl.ds(wid * _NCHUNK, _NCHUNK)], sn_v)

    def _zero(k, c):
        zeros_v[pl.ds(k * 16, 16)] = jnp.zeros((16,), jnp.float32)
        return c
    lax.fori_loop(0, (_NUM_EMBED // 16) // 16, _zero, 0)
    sl = pl.ds(sid * (_NUM_EMBED // 16), _NUM_EMBED // 16)
    pltpu.sync_copy(zeros_v, cnt_sh.at[sl])
    pltpu.sync_copy(zeros_v, wsum_sh.at[sl])

    for k in range(_CHUNK // 16):
        ones_v[pl.ds(k * 16, 16)] = jnp.ones((16,), jnp.float32)

    plsc.subcore_barrier()

    adds = [pltpu.async_copy(ones_v, cnt_sh.at[idx_v.at[j]], addsem,
                             add=True)
            for j in range(_NCHUNK)]
    adds += [pltpu.async_copy(sn_v.at[j], wsum_sh.at[idx_v.at[j]], addsem,
                              add=True)
             for j in range(_NCHUNK)]

    for j in range(_NCHUNK):
        pltpu.async_copy(w_hbm.at[idx_v.at[j]], rows_v, sem).wait()
        pltpu.sync_copy(rows_v, zq_hbm.at[pl.ds(base + j * _CHUNK, _CHUNK)])

    for a in adds:
        a.wait()
    plsc.subcore_barrier()

    @pl.when(sid == 0)
    def _():
        pltpu.sync_copy(cnt_sh, cnt_hbm.at[cid])
        pltpu.sync_copy(wsum_sh, wsum_hbm.at[cid])


def _finalize_body(cnt_ref, wsum_ref, wn_ref, a_ref, loss_ref, ppx_ref):
    a = jnp.sum(a_ref[...])
    wn = wn_ref[...]
    wsum = wsum_ref[0:1, :] + wsum_ref[1:2, :]
    cntf = cnt_ref[0:1, :] + cnt_ref[1:2, :]
    bsum = jnp.sum(wsum * wn)
    csum = jnp.sum(cntf * wn * wn)
    mval = (a - 2.0 * bsum + csum) / jnp.float32(_TOKENS * _EMBED_DIM)
    loss_ref[...] = jnp.reshape(jnp.float32(_BETA) * mval + mval, (1, 1))
    p = cntf * jnp.float32(1.0 / _TOKENS)
    ent = jnp.sum(p * jnp.log(p + 1e-10))
    ppx_ref[...] = jnp.reshape(jnp.exp(-ent), (1, 1))


@functools.lru_cache(maxsize=1)
def _make_sc_gather_hist():
    return pl.kernel(
        _sc_body,
        out_type=(jax.ShapeDtypeStruct((_TOKENS, _EMBED_DIM), jnp.float32),
                  jax.ShapeDtypeStruct((2, _NUM_EMBED), jnp.float32),
                  jax.ShapeDtypeStruct((2, _NUM_EMBED), jnp.float32)),
        mesh=plsc.VectorSubcoreMesh(core_axis_name="c", subcore_axis_name="s",
                                    num_cores=2, num_subcores=16),
        scratch_types=[
            pltpu.VMEM((_NCHUNK, _CHUNK), jnp.int32),
            pltpu.VMEM((_NCHUNK, _CHUNK), jnp.float32),
            pltpu.VMEM((_CHUNK, _EMBED_DIM), jnp.float32),
            pltpu.VMEM((_CHUNK,), jnp.float32),
            pltpu.VMEM((_NUM_EMBED // 16,), jnp.float32),
            pltpu.VMEM_SHARED((_NUM_EMBED,), jnp.float32),
            pltpu.VMEM_SHARED((_NUM_EMBED,), jnp.float32),
            pltpu.SemaphoreType.DMA,
            pltpu.SemaphoreType.DMA,
        ],
    )


def kernel(z, weight):
    b, c, h, w = z.shape
    zp = jnp.transpose(z, (0, 2, 3, 1)).reshape(-1, _EMBED_DIM)

    normed, wncol = pl.pallas_call(
        _normalize_body,
        out_shape=[jax.ShapeDtypeStruct((_NUM_EMBED, _EMBED_DIM),
                                        jnp.bfloat16),
                   jax.ShapeDtypeStruct((_NUM_EMBED, 1), jnp.float32)],
    )(weight)

    n_tiles = _TOKENS // _TM
    idx3, sn3, asum = pl.pallas_call(
        _argmax_body,
        grid=(n_tiles,),
        in_specs=[
            pl.BlockSpec((_TM, _EMBED_DIM), lambda i: (i, 0)),
            pl.BlockSpec((_NUM_EMBED, _EMBED_DIM), lambda i: (0, 0)),
        ],
        out_specs=[
            pl.BlockSpec((1, 1, _TM), lambda i: (i, 0, 0)),
            pl.BlockSpec((1, 1, _TM), lambda i: (i, 0, 0)),
            pl.BlockSpec((1, 1), lambda i: (0, 0)),
        ],
        out_shape=[
            jax.ShapeDtypeStruct((n_tiles, 1, _TM), jnp.int32),
            jax.ShapeDtypeStruct((n_tiles, 1, _TM), jnp.float32),
            jax.ShapeDtypeStruct((1, 1), jnp.float32),
        ],
        scratch_shapes=[pltpu.SMEM((1,), jnp.float32)],
    )(zp, normed)
    idx = idx3.reshape(_TOKENS)

    zq, cnt, wsum = _make_sc_gather_hist()(
        weight, idx.reshape(128, 128), sn3.reshape(128, 128))

    loss, ppx = pl.pallas_call(
        _finalize_body,
        out_shape=[jax.ShapeDtypeStruct((1, 1), jnp.float32),
                   jax.ShapeDtypeStruct((1, 1), jnp.float32)],
    )(cnt, wsum, wncol.reshape(1, _NUM_EMBED), asum)

    z_q_out = zq.reshape(b, h, w, c).transpose(0, 3, 1, 2)
    return z_q_out, loss[0, 0], ppx[0, 0], idx
